# baseline (speedup 1.0000x reference)
import jax
import jax.numpy as jnp
from jax.experimental import pallas as pl
from jax.experimental.pallas import tpu as pltpu
from jax.experimental.pallas import tpu_sc as plsc

B = 4
T = 4096
D = 2048
DFF = 8192
K = 2048
N = B * T
NSEL = B * K
_CH = 256
_EX = D // _CH

_TI = 512


def _select_body(l_ref, zl_ref, mask_ref, gidx_ref):
    l = l_ref[...]
    m = jnp.max(l, axis=1, keepdims=True)
    lse = jnp.log(jnp.sum(jnp.exp(l - m), axis=1, keepdims=True)) + m
    zl_ref[...] = jnp.mean(lse * lse).reshape(1, 1)

    nt = T // _TI
    iota = jax.lax.broadcasted_iota(jnp.int32, (1, T), 1)

    mask_tiles = []
    for it in range(nt):
        li = l[:, it * _TI:(it + 1) * _TI]
        ii = iota[:, it * _TI:(it + 1) * _TI]
        acc = jnp.zeros((B, _TI), jnp.float32)
        for jt in range(nt):
            lj = l[:, jt * _TI:(jt + 1) * _TI]
            jj = iota[:, jt * _TI:(jt + 1) * _TI]
            gt = lj[:, :, None] > li[:, None, :]
            eq = (lj[:, :, None] == li[:, None, :]) & (
                jj[0, :, None] < ii[0, None, :])[None]
            acc += jnp.sum((gt | eq).astype(jnp.float32), axis=1)
        mask_tiles.append((acc < float(K)).astype(jnp.float32))
    mask = jnp.concatenate(mask_tiles, axis=1)
    mask_ref[...] = mask

    lt_intra = (iota[0, :_TI, None] < iota[0, None, :_TI]).astype(
        jnp.float32)[None]
    pos_tiles = []
    offs = jnp.zeros((B, 1), jnp.float32)
    for it in range(nt):
        mt = mask[:, it * _TI:(it + 1) * _TI]
        intra = jnp.sum(mt[:, :, None] * lt_intra, axis=1)
        pos_tiles.append(intra + offs)
        offs = offs + jnp.sum(mt, axis=1, keepdims=True)
    posT = jnp.concatenate(pos_tiles, axis=1)

    boff = jax.lax.broadcasted_iota(jnp.int32, (B, 1), 0).astype(
        jnp.float32) * float(T)
    ciota = jax.lax.broadcasted_iota(jnp.int32, (1, 1, _EX), 2).astype(
        jnp.float32)
    tokf = iota.astype(jnp.float32)
    for pt in range(K // _TI):
        pp = iota[:, pt * _TI:(pt + 1) * _TI].astype(jnp.float32)
        accg = jnp.zeros((B, _TI), jnp.float32)
        for it in range(nt):
            mt = mask[:, it * _TI:(it + 1) * _TI]
            pt_tile = posT[:, it * _TI:(it + 1) * _TI]
            tk = tokf[:, it * _TI:(it + 1) * _TI]
            hit = pt_tile[:, :, None] == pp[0, None, :][None]
            contrib = hit.astype(jnp.float32) * (mt * tk)[:, :, None]
            accg += jnp.sum(contrib, axis=1)
        flat = accg + boff
        gidx_ref[:, pt * _TI:(pt + 1) * _TI, :] = (
            flat[:, :, None] * float(_EX) + ciota).astype(jnp.int32)


def _select(logits2d):
    return pl.pallas_call(
        _select_body,
        out_shape=[
            jax.ShapeDtypeStruct((1, 1), jnp.float32),
            jax.ShapeDtypeStruct((B, T), jnp.float32),
            jax.ShapeDtypeStruct((B, K, _EX), jnp.int32),
        ],
    )(logits2d)


_GW = 128
_NG = NSEL * _EX


def _sc_gather(x_ch, gidx):
    mesh = plsc.VectorSubcoreMesh(core_axis_name="core",
                                  subcore_axis_name="subcore")

    @pl.kernel(out_type=jax.ShapeDtypeStruct((_NG, _CH), jnp.float32),
               mesh=mesh)
    def k(x_hbm, i_hbm, o_hbm):
        def body(i_vmem, o_vmem):
            pltpu.sync_copy(x_hbm.at[i_vmem.at[0]], o_vmem)

        pltpu.emit_pipeline(
            body,
            grid=(_NG // _GW,),
            in_specs=[pl.BlockSpec((1, _GW), lambda i: (0, i))],
            out_specs=[pl.BlockSpec((_GW, _CH), lambda i: (i, 0))],
            core_axis_name=("core", "subcore"),
            dimension_semantics=(pltpu.PARALLEL,),
        )(i_hbm, o_hbm)

    return k(x_ch, gidx)


def _sc_scatter(mlp_ch, gidx):
    mesh = plsc.VectorSubcoreMesh(core_axis_name="core",
                                  subcore_axis_name="subcore")

    @pl.kernel(out_type=jax.ShapeDtypeStruct((N * _EX, _CH), jnp.float32),
               mesh=mesh)
    def k(m_hbm, i_hbm, o_hbm):
        def body(m_vmem, i_vmem):
            pltpu.sync_copy(m_vmem, o_hbm.at[i_vmem.at[0]])

        pltpu.emit_pipeline(
            body,
            grid=(_NG // _GW,),
            in_specs=[
                pl.BlockSpec((_GW, _CH), lambda i: (i, 0)),
                pl.BlockSpec((1, _GW), lambda i: (0, i)),
            ],
            out_specs=[],
            core_axis_name=("core", "subcore"),
            dimension_semantics=(pltpu.PARALLEL,),
        )(m_hbm, i_hbm)

    return k(mlp_ch, gidx)


_MT = 1024
_MF = 1024


def _mlp_body(x_ref, w1_ref, b1_ref, w2_ref, b2_ref, o_ref):
    f = pl.program_id(1)

    @pl.when(f == 0)
    def _():
        o_ref[...] = jnp.broadcast_to(b2_ref[...], (_MT, D))

    h = jnp.dot(x_ref[...].astype(jnp.bfloat16), w1_ref[...],
                preferred_element_type=jnp.float32) + b1_ref[...]
    h = jax.nn.gelu(h)
    o_ref[...] += jnp.dot(h.astype(jnp.bfloat16), w2_ref[...],
                          preferred_element_type=jnp.float32)


def _mlp(xsel, w1, b1_row, w2, b2_row):
    return pl.pallas_call(
        _mlp_body,
        grid=(NSEL // _MT, DFF // _MF),
        in_specs=[
            pl.BlockSpec((_MT, D), lambda t, f: (t, 0)),
            pl.BlockSpec((D, _MF), lambda t, f: (0, f)),
            pl.BlockSpec((1, _MF), lambda t, f: (0, f)),
            pl.BlockSpec((_MF, D), lambda t, f: (f, 0)),
            pl.BlockSpec((1, D), lambda t, f: (0, 0)),
        ],
        out_specs=pl.BlockSpec((_MT, D), lambda t, f: (t, 0)),
        out_shape=jax.ShapeDtypeStruct((NSEL, D), jnp.float32),
    )(xsel, w1, b1_row, w2, b2_row)


_MRT = 512


def _merge_body(x_ref, m_ref, mask_ref, o_ref):
    o_ref[...] = jnp.where(mask_ref[...] > 0.5, m_ref[...], x_ref[...])


def _merge(x2d, staged, maskN):
    return pl.pallas_call(
        _merge_body,
        grid=(N // _MRT,),
        in_specs=[
            pl.BlockSpec((_MRT, D), lambda i: (i, 0)),
            pl.BlockSpec((_MRT, D), lambda i: (i, 0)),
            pl.BlockSpec((_MRT, 1), lambda i: (i, 0)),
        ],
        out_specs=pl.BlockSpec((_MRT, D), lambda i: (i, 0)),
        out_shape=jax.ShapeDtypeStruct((N, D), jnp.float32),
    )(x2d, staged, maskN)


def kernel(x, Wr, W1, b1, W2, b2):
    x2d = x.reshape(N, D)
    logits = x2d @ Wr
    zl, mask2d, gidx_bpe = _select(logits.reshape(B, T))
    maskN = mask2d.reshape(N, 1)
    gidx = gidx_bpe.reshape(1, _NG)
    xsel = _sc_gather(x2d.reshape(N * _EX, _CH), gidx)
    mlp_out = _mlp(xsel.reshape(NSEL, D), W1.astype(jnp.bfloat16),
                   b1.reshape(1, DFF), W2.astype(jnp.bfloat16),
                   b2.reshape(1, D))
    staged = _sc_scatter(mlp_out.reshape(_NG, _CH), gidx)
    out2d = _merge(x2d, staged.reshape(N, D), maskN)
    return out2d.reshape(B, T, D), zl.reshape(())


# speedup vs baseline: 1.0664x; 1.0664x over previous
import jax
import jax.numpy as jnp
from jax.experimental import pallas as pl
from jax.experimental.pallas import tpu as pltpu
from jax.experimental.pallas import tpu_sc as plsc

B = 4
T = 4096
D = 2048
DFF = 8192
K = 2048
N = B * T
NSEL = B * K
_CH = 256
_EX = D // _CH

_TI = 512


def _select_body(l_ref, zl_ref, gidx_ref):
    l = l_ref[...]
    m = jnp.max(l, axis=1, keepdims=True)
    lse = jnp.log(jnp.sum(jnp.exp(l - m), axis=1, keepdims=True)) + m
    zl_ref[...] = jnp.mean(lse * lse).reshape(1, 1)

    nt = T // _TI
    iota = jax.lax.broadcasted_iota(jnp.int32, (1, T), 1)

    mask_tiles = []
    for it in range(nt):
        li = l[:, it * _TI:(it + 1) * _TI]
        ii = iota[:, it * _TI:(it + 1) * _TI]
        acc = jnp.zeros((B, _TI), jnp.float32)
        for jt in range(nt):
            lj = l[:, jt * _TI:(jt + 1) * _TI]
            jj = iota[:, jt * _TI:(jt + 1) * _TI]
            gt = lj[:, :, None] > li[:, None, :]
            eq = (lj[:, :, None] == li[:, None, :]) & (
                jj[0, :, None] < ii[0, None, :])[None]
            acc += jnp.sum((gt | eq).astype(jnp.float32), axis=1)
        mask_tiles.append((acc < float(K)).astype(jnp.float32))
    mask = jnp.concatenate(mask_tiles, axis=1)

    lt_intra = (iota[0, :_TI, None] < iota[0, None, :_TI]).astype(
        jnp.float32)[None]
    pos_tiles = []
    offs = jnp.zeros((B, 1), jnp.float32)
    for it in range(nt):
        mt = mask[:, it * _TI:(it + 1) * _TI]
        intra = jnp.sum(mt[:, :, None] * lt_intra, axis=1)
        pos_tiles.append(intra + offs)
        offs = offs + jnp.sum(mt, axis=1, keepdims=True)
    posT = jnp.concatenate(pos_tiles, axis=1)

    boff = jax.lax.broadcasted_iota(jnp.int32, (B, 1), 0).astype(
        jnp.float32) * float(T)
    ciota = jax.lax.broadcasted_iota(jnp.int32, (1, 1, _EX), 2).astype(
        jnp.float32)
    tokf = iota.astype(jnp.float32)
    for pt in range(K // _TI):
        pp = iota[:, pt * _TI:(pt + 1) * _TI].astype(jnp.float32)
        accg = jnp.zeros((B, _TI), jnp.float32)
        for it in range(nt):
            mt = mask[:, it * _TI:(it + 1) * _TI]
            pt_tile = posT[:, it * _TI:(it + 1) * _TI]
            tk = tokf[:, it * _TI:(it + 1) * _TI]
            hit = pt_tile[:, :, None] == pp[0, None, :][None]
            contrib = hit.astype(jnp.float32) * (mt * tk)[:, :, None]
            accg += jnp.sum(contrib, axis=1)
        flat = accg + boff
        gidx_ref[:, pt * _TI:(pt + 1) * _TI, :] = (
            flat[:, :, None] * float(_EX) + ciota).astype(jnp.int32)


def _select(logits2d):
    return pl.pallas_call(
        _select_body,
        out_shape=[
            jax.ShapeDtypeStruct((1, 1), jnp.float32),
            jax.ShapeDtypeStruct((B, K, _EX), jnp.int32),
        ],
    )(logits2d)


_GW = 128
_NB = K * _EX
_NG = NSEL * _EX

_MESH = plsc.VectorSubcoreMesh(core_axis_name="core",
                               subcore_axis_name="subcore")


def _sc_gather(x_ch, gidx_b):
    @pl.kernel(out_type=jax.ShapeDtypeStruct((_NB, _CH), jnp.float32),
               mesh=_MESH)
    def k(x_hbm, i_hbm, o_hbm):
        def body(i_vmem, o_vmem):
            pltpu.sync_copy(x_hbm.at[i_vmem.at[0]], o_vmem)

        pltpu.emit_pipeline(
            body,
            grid=(_NB // _GW,),
            in_specs=[pl.BlockSpec((1, _GW), lambda i: (0, i))],
            out_specs=[pl.BlockSpec((_GW, _CH), lambda i: (i, 0))],
            core_axis_name=("core", "subcore"),
            dimension_semantics=(pltpu.PARALLEL,),
        )(i_hbm, o_hbm)

    return k(x_ch, gidx_b)


def _sc_scatter(mlp_ch, gidx_b, out_ref):
    @pl.kernel(out_type=(), mesh=_MESH)
    def k(m_hbm, i_hbm, o_hbm):
        def body(m_vmem, i_vmem):
            pltpu.sync_copy(m_vmem, o_hbm.at[i_vmem.at[0]])

        pltpu.emit_pipeline(
            body,
            grid=(_NB // _GW,),
            in_specs=[
                pl.BlockSpec((_GW, _CH), lambda i: (i, 0)),
                pl.BlockSpec((1, _GW), lambda i: (0, i)),
            ],
            out_specs=[],
            core_axis_name=("core", "subcore"),
            dimension_semantics=(pltpu.PARALLEL,),
        )(m_hbm, i_hbm)

    return k(mlp_ch, gidx_b, out_ref)


_MT = 1024
_MF = 1024


def _mlp_body(x_ref, w1_ref, b1_ref, w2_ref, b2_ref, o_ref):
    f = pl.program_id(1)

    @pl.when(f == 0)
    def _():
        o_ref[...] = jnp.broadcast_to(b2_ref[...], (_MT, D))

    h = jnp.dot(x_ref[...].astype(jnp.bfloat16), w1_ref[...],
                preferred_element_type=jnp.float32) + b1_ref[...]
    h = jax.nn.gelu(h)
    o_ref[...] += jnp.dot(h.astype(jnp.bfloat16), w2_ref[...],
                          preferred_element_type=jnp.float32)


def _mlp(xsel, w1, b1_row, w2, b2_row):
    return pl.pallas_call(
        _mlp_body,
        grid=(K // _MT, DFF // _MF),
        in_specs=[
            pl.BlockSpec((_MT, D), lambda t, f: (t, 0)),
            pl.BlockSpec((D, _MF), lambda t, f: (0, f)),
            pl.BlockSpec((1, _MF), lambda t, f: (0, f)),
            pl.BlockSpec((_MF, D), lambda t, f: (f, 0)),
            pl.BlockSpec((1, D), lambda t, f: (0, 0)),
        ],
        out_specs=pl.BlockSpec((_MT, D), lambda t, f: (t, 0)),
        out_shape=jax.ShapeDtypeStruct((K, D), jnp.float32),
    )(xsel, w1, b1_row, w2, b2_row)


def kernel(x, Wr, W1, b1, W2, b2):
    x2d = x.reshape(N, D)
    logits = x2d @ Wr
    zl, gidx_bpe = _select(logits.reshape(B, T))
    x_ch = x2d.reshape(N * _EX, _CH)
    w1 = W1.astype(jnp.bfloat16)
    w2 = W2.astype(jnp.bfloat16)
    b1r = b1.reshape(1, DFF)
    b2r = b2.reshape(1, D)
    out_ref = jax.new_ref(x_ch)
    for b in range(B):
        gidx_b = gidx_bpe[b].reshape(1, _NB)
        xsel = _sc_gather(x_ch, gidx_b)
        mlp_out = _mlp(xsel.reshape(K, D), w1, b1r, w2, b2r)
        _sc_scatter(mlp_out.reshape(_NB, _CH), gidx_b, out_ref)
    return out_ref[...].reshape(B, T, D), zl.reshape(())


# speedup vs baseline: 1.5206x; 1.4259x over previous
import jax
import jax.numpy as jnp
from jax.experimental import pallas as pl
from jax.experimental.pallas import tpu as pltpu
from jax.experimental.pallas import tpu_sc as plsc

B = 4
T = 4096
D = 2048
DFF = 8192
K = 2048
N = B * T
NSEL = B * K
_CW = 32
_NCB = K // _CW
_NWK = 32

_TI = 512


def _select_body(l_ref, zl_ref, gidx_ref):
    l = l_ref[...]
    m = jnp.max(l, axis=1, keepdims=True)
    lse = jnp.log(jnp.sum(jnp.exp(l - m), axis=1, keepdims=True)) + m
    zl_ref[...] = jnp.mean(lse * lse).reshape(1, 1)

    nt = T // _TI
    iota = jax.lax.broadcasted_iota(jnp.int32, (1, T), 1)

    mask_tiles = []
    for it in range(nt):
        li = l[:, it * _TI:(it + 1) * _TI]
        ii = iota[:, it * _TI:(it + 1) * _TI]
        acc = jnp.zeros((B, _TI), jnp.float32)
        for jt in range(nt):
            lj = l[:, jt * _TI:(jt + 1) * _TI]
            jj = iota[:, jt * _TI:(jt + 1) * _TI]
            gt = lj[:, :, None] > li[:, None, :]
            eq = (lj[:, :, None] == li[:, None, :]) & (
                jj[0, :, None] < ii[0, None, :])[None]
            acc += jnp.sum((gt | eq).astype(jnp.float32), axis=1)
        mask_tiles.append((acc < float(K)).astype(jnp.float32))
    mask = jnp.concatenate(mask_tiles, axis=1)

    lt_intra = (iota[0, :_TI, None] < iota[0, None, :_TI]).astype(
        jnp.float32)[None]
    pos_tiles = []
    offs = jnp.zeros((B, 1), jnp.float32)
    for it in range(nt):
        mt = mask[:, it * _TI:(it + 1) * _TI]
        intra = jnp.sum(mt[:, :, None] * lt_intra, axis=1)
        pos_tiles.append(intra + offs)
        offs = offs + jnp.sum(mt, axis=1, keepdims=True)
    posT = jnp.concatenate(pos_tiles, axis=1)

    boff = jax.lax.broadcasted_iota(jnp.int32, (B, 1), 0).astype(
        jnp.float32) * float(T)
    tokf = iota.astype(jnp.float32)
    for pt in range(K // _TI):
        pp = iota[:, pt * _TI:(pt + 1) * _TI].astype(jnp.float32)
        accg = jnp.zeros((B, _TI), jnp.float32)
        for it in range(nt):
            mt = mask[:, it * _TI:(it + 1) * _TI]
            pt_tile = posT[:, it * _TI:(it + 1) * _TI]
            tk = tokf[:, it * _TI:(it + 1) * _TI]
            hit = pt_tile[:, :, None] == pp[0, None, :][None]
            contrib = hit.astype(jnp.float32) * (mt * tk)[:, :, None]
            accg += jnp.sum(contrib, axis=1)
        gidx_ref[:, pt * _TI:(pt + 1) * _TI] = (accg + boff).astype(jnp.int32)


def _select(logits2d):
    return pl.pallas_call(
        _select_body,
        out_shape=[
            jax.ShapeDtypeStruct((1, 1), jnp.float32),
            jax.ShapeDtypeStruct((B, K), jnp.int32),
        ],
    )(logits2d)


_MESH = plsc.VectorSubcoreMesh(core_axis_name="core",
                               subcore_axis_name="subcore")
_SC_SCRATCH = [
    pltpu.VMEM((1, _CW), jnp.int32),
    pltpu.VMEM((_CW, D), jnp.float32),
    pltpu.SemaphoreType.DMA,
]


def _sc_gather(x2d, idx2d_b):
    @pl.kernel(out_type=jax.ShapeDtypeStruct((K, D), jnp.float32),
               mesh=_MESH, scratch_types=_SC_SCRATCH)
    def k(x_hbm, i_hbm, o_hbm, idx_v, rows_v, sem):
        wid = jax.lax.axis_index("core") * 16 + jax.lax.axis_index("subcore")

        @pl.loop(0, _NCB // _NWK)
        def _(j):
            c = wid * (_NCB // _NWK) + j
            pltpu.sync_copy(i_hbm.at[pl.ds(c, 1)], idx_v)
            pltpu.async_copy(x_hbm.at[idx_v.at[0]], rows_v, sem).wait()
            pltpu.sync_copy(rows_v, o_hbm.at[pl.ds(c * _CW, _CW)])

    return k(x2d, idx2d_b)


def _sc_scatter(mlp, idx2d_b, out_ref):
    @pl.kernel(out_type=(), mesh=_MESH, scratch_types=_SC_SCRATCH)
    def k(m_hbm, i_hbm, o_hbm, idx_v, rows_v, sem):
        wid = jax.lax.axis_index("core") * 16 + jax.lax.axis_index("subcore")

        @pl.loop(0, _NCB // _NWK)
        def _(j):
            c = wid * (_NCB // _NWK) + j
            pltpu.sync_copy(i_hbm.at[pl.ds(c, 1)], idx_v)
            pltpu.sync_copy(m_hbm.at[pl.ds(c * _CW, _CW)], rows_v)
            pltpu.async_copy(rows_v, o_hbm.at[idx_v.at[0]], sem).wait()

    return k(mlp, idx2d_b, out_ref)


_MT = 1024
_MF = 512


def _mlp_body(x_ref, w1_ref, b1_ref, w2_ref, b2_ref, o_ref):
    f = pl.program_id(1)

    @pl.when(f == 0)
    def _():
        o_ref[...] = jnp.broadcast_to(b2_ref[...], (_MT, D))

    h = jnp.dot(x_ref[...].astype(jnp.bfloat16),
                w1_ref[...].astype(jnp.bfloat16),
                preferred_element_type=jnp.float32) + b1_ref[...]
    h = jax.nn.gelu(h)
    o_ref[...] += jnp.dot(h.astype(jnp.bfloat16),
                          w2_ref[...].astype(jnp.bfloat16),
                          preferred_element_type=jnp.float32)


def _mlp(xsel, w1, b1_row, w2, b2_row):
    return pl.pallas_call(
        _mlp_body,
        grid=(K // _MT, DFF // _MF),
        in_specs=[
            pl.BlockSpec((_MT, D), lambda t, f: (t, 0)),
            pl.BlockSpec((D, _MF), lambda t, f: (0, f)),
            pl.BlockSpec((1, _MF), lambda t, f: (0, f)),
            pl.BlockSpec((_MF, D), lambda t, f: (f, 0)),
            pl.BlockSpec((1, D), lambda t, f: (0, 0)),
        ],
        out_specs=pl.BlockSpec((_MT, D), lambda t, f: (t, 0)),
        out_shape=jax.ShapeDtypeStruct((K, D), jnp.float32),
    )(xsel, w1, b1_row, w2, b2_row)


def kernel(x, Wr, W1, b1, W2, b2):
    x2d = x.reshape(N, D)
    logits = x2d @ Wr
    zl, gidx = _select(logits.reshape(B, T))
    b1r = b1.reshape(1, DFF)
    b2r = b2.reshape(1, D)
    out_ref = jax.new_ref(x2d)
    for b in range(B):
        idx2d_b = gidx[b].reshape(_NCB, _CW)
        xsel = _sc_gather(x2d, idx2d_b)
        mlp_out = _mlp(xsel, W1, b1r, W2, b2r)
        _sc_scatter(mlp_out, idx2d_b, out_ref)
    return out_ref[...].reshape(B, T, D), zl.reshape(())
